# baseline (device time: 34034 ns/iter reference)
import jax
import jax.numpy as jnp
from jax import lax
from jax.experimental import pallas as pl
from jax.experimental.pallas import tpu as pltpu

N_DEV = 16
B, SQ, SKV = 2, 256, 256
HQ_TOT, DH = 64, 64
H_LOC = HQ_TOT // N_DEV
BLK = 64
D_MODEL = 512
D_HEADS = H_LOC * DH
ROWS = B * SQ
SEG = ROWS // N_DEV


def kernel(x, Wq, K_ext, V_ext, Wo):
    K2 = K_ext.reshape(B, SKV, D_HEADS)
    V2 = V_ext.reshape(B, SKV, D_HEADS)

    def body(x_ref, wq_ref, k_ref, v_ref, wo_ref, out_ref,
             pbf_ref, stage_ref, gbuf_ref, allout_ref,
             send1, recv1, send2, recv2):
        my = lax.axis_index("i")

        wq = wq_ref[:, pl.ds(my * D_HEADS, D_HEADS)].astype(jnp.bfloat16)
        wo = wo_ref[pl.ds(my * D_HEADS, D_HEADS), :].astype(jnp.bfloat16)

        row_blk = lax.broadcasted_iota(jnp.int32, (SQ, SKV), 0) // BLK
        col_blk = lax.broadcasted_iota(jnp.int32, (SQ, SKV), 1) // BLK
        mask = col_blk <= row_blk

        xx = x_ref[...].reshape(ROWS, D_MODEL).astype(jnp.bfloat16)
        q_all = jnp.dot(xx, wq, preferred_element_type=jnp.float32)

        ctx_parts = []
        for b in range(B):
            qb = q_all[b * SQ:(b + 1) * SQ]
            kb = k_ref[b].astype(jnp.bfloat16)
            vb = v_ref[b].astype(jnp.bfloat16)
            heads = []
            for h in range(H_LOC):
                qh = qb[:, h * DH:(h + 1) * DH].astype(jnp.bfloat16)
                kh = kb[:, h * DH:(h + 1) * DH]
                vh = vb[:, h * DH:(h + 1) * DH]
                s = lax.dot_general(
                    qh, kh, (((1,), (1,)), ((), ())),
                    preferred_element_type=jnp.float32,
                ) * 0.125
                s = jnp.where(mask, s, -1e9)
                m = jnp.max(s, axis=-1, keepdims=True)
                w = jnp.exp(s - m)
                w = w / jnp.sum(w, axis=-1, keepdims=True)
                heads.append(jnp.dot(
                    w.astype(jnp.bfloat16), vh,
                    preferred_element_type=jnp.float32,
                ))
            ctx_parts.append(jnp.concatenate(heads, axis=1))
        ctx_all = jnp.concatenate(ctx_parts, axis=0).astype(jnp.bfloat16)
        part = jnp.dot(ctx_all, wo, preferred_element_type=jnp.float32)
        pbf_ref[...] = part.astype(jnp.bfloat16)

        barrier = pltpu.get_barrier_semaphore()
        for d in range(1, N_DEV):
            pl.semaphore_signal(
                barrier, inc=1,
                device_id=((my + d) % N_DEV,),
                device_id_type=pl.DeviceIdType.MESH,
            )
        pl.semaphore_wait(barrier, N_DEV - 1)

        sends1 = []
        for d in range(1, N_DEV):
            p = (my + d) % N_DEV
            r = pltpu.make_async_remote_copy(
                src_ref=pbf_ref.at[pl.ds(p * SEG, SEG)],
                dst_ref=stage_ref.at[pl.ds(my * SEG, SEG)],
                send_sem=send1.at[p],
                recv_sem=recv1.at[my],
                device_id=(p,),
                device_id_type=pl.DeviceIdType.MESH,
            )
            r.start()
            sends1.append(r)
        stage_ref[pl.ds(my * SEG, SEG), :] = pbf_ref[pl.ds(my * SEG, SEG), :]
        for d in range(1, N_DEV):
            q_src = (my + d) % N_DEV
            pltpu.make_async_remote_copy(
                src_ref=pbf_ref.at[pl.ds(0, SEG)],
                dst_ref=stage_ref.at[pl.ds(q_src * SEG, SEG)],
                send_sem=send1.at[q_src],
                recv_sem=recv1.at[q_src],
                device_id=(q_src,),
                device_id_type=pl.DeviceIdType.MESH,
            ).wait_recv()
        for r in sends1:
            r.wait_send()

        s = stage_ref[...].astype(jnp.float32)
        seg_sum = s.reshape(N_DEV, SEG, D_MODEL).sum(axis=0)
        gbuf_ref[...] = seg_sum.astype(jnp.bfloat16)

        sends2 = []
        for d in range(1, N_DEV):
            p = (my + d) % N_DEV
            r = pltpu.make_async_remote_copy(
                src_ref=gbuf_ref,
                dst_ref=allout_ref.at[pl.ds(my * SEG, SEG)],
                send_sem=send2.at[p],
                recv_sem=recv2.at[my],
                device_id=(p,),
                device_id_type=pl.DeviceIdType.MESH,
            )
            r.start()
            sends2.append(r)
        allout_ref[pl.ds(my * SEG, SEG), :] = gbuf_ref[...]
        for d in range(1, N_DEV):
            q_src = (my + d) % N_DEV
            pltpu.make_async_remote_copy(
                src_ref=gbuf_ref,
                dst_ref=allout_ref.at[pl.ds(q_src * SEG, SEG)],
                send_sem=send2.at[q_src],
                recv_sem=recv2.at[q_src],
                device_id=(q_src,),
                device_id_type=pl.DeviceIdType.MESH,
            ).wait_recv()
        for r in sends2:
            r.wait_send()

        out_ref[...] = allout_ref[...].astype(jnp.float32).reshape(
            B, SQ, D_MODEL)

    return pl.pallas_call(
        body,
        out_shape=jax.ShapeDtypeStruct((B, SQ, D_MODEL), jnp.float32),
        in_specs=[pl.BlockSpec(memory_space=pltpu.VMEM)] * 5,
        out_specs=pl.BlockSpec(memory_space=pltpu.VMEM),
        scratch_shapes=[
            pltpu.VMEM((ROWS, D_MODEL), jnp.bfloat16),
            pltpu.VMEM((ROWS, D_MODEL), jnp.bfloat16),
            pltpu.VMEM((SEG, D_MODEL), jnp.bfloat16),
            pltpu.VMEM((ROWS, D_MODEL), jnp.bfloat16),
            pltpu.SemaphoreType.DMA((N_DEV,)),
            pltpu.SemaphoreType.DMA((N_DEV,)),
            pltpu.SemaphoreType.DMA((N_DEV,)),
            pltpu.SemaphoreType.DMA((N_DEV,)),
        ],
        compiler_params=pltpu.CompilerParams(collective_id=0),
    )(x, Wq, K2, V2, Wo)


# device time: 33935 ns/iter; 1.0029x vs baseline; 1.0029x over previous
import jax
import jax.numpy as jnp
from jax import lax
from jax.experimental import pallas as pl
from jax.experimental.pallas import tpu as pltpu

N_DEV = 16
B, SQ, SKV = 2, 256, 256
HQ_TOT, DH = 64, 64
H_LOC = HQ_TOT // N_DEV
BLK = 64
D_MODEL = 512
D_HEADS = H_LOC * DH
ROWS = B * SQ
SEG = ROWS // N_DEV


def kernel(x, Wq, K_ext, V_ext, Wo):
    K2 = K_ext.reshape(B, SKV, D_HEADS)
    V2 = V_ext.reshape(B, SKV, D_HEADS)

    def body(x_ref, wq_ref, k_ref, v_ref, wo_ref, out_ref,
             pbf_ref, stage_ref, gbuf_ref, allout_ref,
             wq_vmem, wo_vmem, wdma_sems,
             send1, recv1, send2, recv2):
        my = lax.axis_index("i")

        wq_dma = pltpu.make_async_copy(
            wq_ref.at[:, pl.ds(my * D_HEADS, D_HEADS)], wq_vmem,
            wdma_sems.at[0])
        wo_dma = pltpu.make_async_copy(
            wo_ref.at[pl.ds(my * D_HEADS, D_HEADS), :], wo_vmem,
            wdma_sems.at[1])
        wq_dma.start()
        wo_dma.start()

        row_blk = lax.broadcasted_iota(jnp.int32, (SQ, SKV), 0) // BLK
        col_blk = lax.broadcasted_iota(jnp.int32, (SQ, SKV), 1) // BLK
        mask = col_blk <= row_blk

        xx = x_ref[...].reshape(ROWS, D_MODEL).astype(jnp.bfloat16)
        wq_dma.wait()
        wq = wq_vmem[...].astype(jnp.bfloat16)
        q_all = jnp.dot(xx, wq, preferred_element_type=jnp.float32)

        ctx_parts = []
        for b in range(B):
            qb = q_all[b * SQ:(b + 1) * SQ]
            kb = k_ref[b].astype(jnp.bfloat16)
            vb = v_ref[b].astype(jnp.bfloat16)
            heads = []
            for h in range(H_LOC):
                qh = qb[:, h * DH:(h + 1) * DH].astype(jnp.bfloat16)
                kh = kb[:, h * DH:(h + 1) * DH]
                vh = vb[:, h * DH:(h + 1) * DH]
                s = lax.dot_general(
                    qh, kh, (((1,), (1,)), ((), ())),
                    preferred_element_type=jnp.float32,
                ) * 0.125
                w = jnp.where(mask, jnp.exp(s), 0.0)
                w = w / jnp.sum(w, axis=-1, keepdims=True)
                heads.append(jnp.dot(
                    w.astype(jnp.bfloat16), vh,
                    preferred_element_type=jnp.float32,
                ))
            ctx_parts.append(jnp.concatenate(heads, axis=1))
        ctx_all = jnp.concatenate(ctx_parts, axis=0).astype(jnp.bfloat16)
        wo_dma.wait()
        wo = wo_vmem[...].astype(jnp.bfloat16)
        part = jnp.dot(ctx_all, wo, preferred_element_type=jnp.float32)
        pbf_ref[...] = part.astype(jnp.bfloat16)

        barrier = pltpu.get_barrier_semaphore()
        for d in range(1, N_DEV):
            pl.semaphore_signal(
                barrier, inc=1,
                device_id=((my + d) % N_DEV,),
                device_id_type=pl.DeviceIdType.MESH,
            )
        pl.semaphore_wait(barrier, N_DEV - 1)

        sends1 = []
        for d in range(1, N_DEV):
            p = (my + d) % N_DEV
            r = pltpu.make_async_remote_copy(
                src_ref=pbf_ref.at[pl.ds(p * SEG, SEG)],
                dst_ref=stage_ref.at[pl.ds(my * SEG, SEG)],
                send_sem=send1.at[p],
                recv_sem=recv1.at[my],
                device_id=(p,),
                device_id_type=pl.DeviceIdType.MESH,
            )
            r.start()
            sends1.append(r)
        stage_ref[pl.ds(my * SEG, SEG), :] = pbf_ref[pl.ds(my * SEG, SEG), :]
        for d in range(1, N_DEV):
            q_src = (my + d) % N_DEV
            pltpu.make_async_remote_copy(
                src_ref=pbf_ref.at[pl.ds(0, SEG)],
                dst_ref=stage_ref.at[pl.ds(q_src * SEG, SEG)],
                send_sem=send1.at[q_src],
                recv_sem=recv1.at[q_src],
                device_id=(q_src,),
                device_id_type=pl.DeviceIdType.MESH,
            ).wait_recv()
        for r in sends1:
            r.wait_send()

        s = stage_ref[...].astype(jnp.float32)
        seg_sum = s.reshape(N_DEV, SEG, D_MODEL).sum(axis=0)
        gbuf_ref[...] = seg_sum.astype(jnp.bfloat16)

        sends2 = []
        for d in range(1, N_DEV):
            p = (my + d) % N_DEV
            r = pltpu.make_async_remote_copy(
                src_ref=gbuf_ref,
                dst_ref=allout_ref.at[pl.ds(my * SEG, SEG)],
                send_sem=send2.at[p],
                recv_sem=recv2.at[my],
                device_id=(p,),
                device_id_type=pl.DeviceIdType.MESH,
            )
            r.start()
            sends2.append(r)
        allout_ref[pl.ds(my * SEG, SEG), :] = gbuf_ref[...]
        for d in range(1, N_DEV):
            q_src = (my + d) % N_DEV
            pltpu.make_async_remote_copy(
                src_ref=gbuf_ref,
                dst_ref=allout_ref.at[pl.ds(q_src * SEG, SEG)],
                send_sem=send2.at[q_src],
                recv_sem=recv2.at[q_src],
                device_id=(q_src,),
                device_id_type=pl.DeviceIdType.MESH,
            ).wait_recv()
        for r in sends2:
            r.wait_send()

        out_ref[...] = allout_ref[...].astype(jnp.float32).reshape(
            B, SQ, D_MODEL)

    return pl.pallas_call(
        body,
        out_shape=jax.ShapeDtypeStruct((B, SQ, D_MODEL), jnp.float32),
        in_specs=[
            pl.BlockSpec(memory_space=pltpu.VMEM),
            pl.BlockSpec(memory_space=pl.ANY),
            pl.BlockSpec(memory_space=pltpu.VMEM),
            pl.BlockSpec(memory_space=pltpu.VMEM),
            pl.BlockSpec(memory_space=pl.ANY),
        ],
        out_specs=pl.BlockSpec(memory_space=pltpu.VMEM),
        scratch_shapes=[
            pltpu.VMEM((ROWS, D_MODEL), jnp.bfloat16),
            pltpu.VMEM((ROWS, D_MODEL), jnp.bfloat16),
            pltpu.VMEM((SEG, D_MODEL), jnp.bfloat16),
            pltpu.VMEM((ROWS, D_MODEL), jnp.bfloat16),
            pltpu.VMEM((D_MODEL, D_HEADS), jnp.float32),
            pltpu.VMEM((D_HEADS, D_MODEL), jnp.float32),
            pltpu.SemaphoreType.DMA((2,)),
            pltpu.SemaphoreType.DMA((N_DEV,)),
            pltpu.SemaphoreType.DMA((N_DEV,)),
            pltpu.SemaphoreType.DMA((N_DEV,)),
            pltpu.SemaphoreType.DMA((N_DEV,)),
        ],
        compiler_params=pltpu.CompilerParams(collective_id=0),
    )(x, Wq, K2, V2, Wo)
